# baseline (speedup 1.0000x reference)
import functools

import jax
import jax.numpy as jnp
from jax import lax
from jax.experimental import pallas as pl
from jax.experimental.pallas import tpu as pltpu
from jax.experimental.pallas import tpu_sc as plsc

V = 1000000
D = 64
B = 16384
NC = 2
NS = 16
NW = NC * NS
CHUNK = B // NW
L = 16


@functools.partial(
    pl.kernel,
    mesh=plsc.VectorSubcoreMesh(core_axis_name="c", subcore_axis_name="s"),
    out_type=jax.ShapeDtypeStruct((2 * B,), jnp.float32),
    compiler_params=pltpu.CompilerParams(
        needs_layout_passes=False, use_tc_tiling_on_sc=False
    ),
    scratch_types=[
        pltpu.VMEM((CHUNK,), jnp.int32),
        pltpu.VMEM((CHUNK,), jnp.int32),
        pltpu.VMEM((CHUNK,), jnp.int32),
        pltpu.VMEM((CHUNK, D), jnp.float32),
        pltpu.VMEM((CHUNK, D), jnp.float32),
        pltpu.VMEM((CHUNK, D), jnp.float32),
        pltpu.VMEM((CHUNK,), jnp.float32),
        pltpu.VMEM((CHUNK,), jnp.float32),
        pltpu.SemaphoreType.DMA,
    ],
)
def _sc_dots(iv_hbm, ov_hbm, iw_hbm, ow_hbm, nw_hbm, out_hbm,
             idxi, idxo, idxn, rows_iv, rows_ov, rows_nv, odot, ndot, sem):
    wid = lax.axis_index("s") * NC + lax.axis_index("c")
    base = wid * CHUNK
    pltpu.sync_copy(iw_hbm.at[pl.ds(base, CHUNK)], idxi)
    pltpu.sync_copy(ow_hbm.at[pl.ds(base, CHUNK)], idxo)
    pltpu.sync_copy(nw_hbm.at[pl.ds(base, CHUNK)], idxn)
    ci = pltpu.async_copy(iv_hbm.at[idxi], rows_iv, sem)
    co = pltpu.async_copy(ov_hbm.at[idxo], rows_ov, sem)
    cn = pltpu.async_copy(ov_hbm.at[idxn], rows_nv, sem)
    ci.wait()
    co.wait()
    cn.wait()

    lane = lax.iota(jnp.int32, L)
    last_lane = lane == (L - 1)

    def row(r, _):
        acc_o = jnp.zeros((L,), jnp.float32)
        acc_n = jnp.zeros((L,), jnp.float32)
        for k in range(D // L):
            ivk = rows_iv[r, pl.ds(k * L, L)]
            acc_o = acc_o + ivk * rows_ov[r, pl.ds(k * L, L)]
            acc_n = acc_n + ivk * rows_nv[r, pl.ds(k * L, L)]
        ridx = jnp.full((L,), r, jnp.int32)
        plsc.store_scatter(odot, [ridx], plsc.cumsum(acc_o), mask=last_lane)
        plsc.store_scatter(ndot, [ridx], plsc.cumsum(acc_n), mask=last_lane)
        return 0

    lax.fori_loop(0, CHUNK, row, 0)
    pltpu.sync_copy(odot, out_hbm.at[pl.ds(base, CHUNK)])
    pltpu.sync_copy(ndot, out_hbm.at[pl.ds(B + base, CHUNK)])


def _tc_loss_body(d_ref, out_ref):
    o = d_ref[0:1, :]
    n = d_ref[1:2, :]
    loss = jax.nn.log_sigmoid(o) + jax.nn.log_sigmoid(-n)
    out_ref[...] = jnp.full((1, 1), -jnp.sum(loss) / B, jnp.float32)


_tc_loss = pl.pallas_call(
    _tc_loss_body,
    out_shape=jax.ShapeDtypeStruct((1, 1), jnp.float32),
)


def kernel(ivectors, ovectors, iword, owords, nwords):
    iw = iword.astype(jnp.int32)
    ow = owords.astype(jnp.int32)
    nw = nwords.astype(jnp.int32)
    dots = _sc_dots(ivectors, ovectors, iw, ow, nw)
    loss = _tc_loss(dots.reshape(2, B))
    return loss[0, 0]
